# Initial kernel scaffold; baseline (speedup 1.0000x reference)
import jax
import jax.numpy as jnp
from jax.experimental import pallas as pl


def kernel(x, kernels, rec_kernels, biases):
    raise NotImplementedError("write your pallas kernel here")


# speedup vs baseline: 3.9779x; 3.9779x over previous
import functools

import jax
import jax.numpy as jnp
from jax.experimental import pallas as pl
from jax.experimental.pallas import tpu as pltpu


def _gru_layer_body(x_ref, w_ref, rw_ref, b_ref, y_ref, hT_ref, gx_ref, h_ref,
                    *, TB, BB, U):
    t = pl.program_id(1)

    @pl.when(t == 0)
    def _init():
        h_ref[...] = jnp.zeros_like(h_ref)

    D = x_ref.shape[-1]
    xb = x_ref[...].reshape(TB * BB, D)
    gx = jnp.dot(xb, w_ref[...], preferred_element_type=jnp.float32) + b_ref[0]
    gx_ref[...] = gx.reshape(TB, BB, 3 * U)
    rb = b_ref[1]

    def step(i, _):
        h = h_ref[...]
        gxt = gx_ref[i]
        gh = jnp.dot(h, rw_ref[...], preferred_element_type=jnp.float32) + rb
        z = jax.nn.sigmoid(gxt[:, :U] + gh[:, :U])
        r = jax.nn.sigmoid(gxt[:, U:2 * U] + gh[:, U:2 * U])
        hh = jnp.tanh(gxt[:, 2 * U:] + r * gh[:, 2 * U:])
        hn = z * h + (1.0 - z) * hh
        h_ref[...] = hn
        y_ref[i] = hn.astype(y_ref.dtype)
        return None

    jax.lax.fori_loop(0, TB, step, None)
    hT_ref[...] = h_ref[...].astype(hT_ref.dtype)


def _gru_layer(xt, w, rw, b, *, interpret=False):
    T, B, D = xt.shape
    U = rw.shape[0]
    NB = 2
    BB = B // NB
    TB = 16
    NT = T // TB

    body = functools.partial(_gru_layer_body, TB=TB, BB=BB, U=U)
    yt, hT = pl.pallas_call(
        body,
        grid=(NB, NT),
        in_specs=[
            pl.BlockSpec((TB, BB, D), lambda bi, ti: (ti, bi, 0)),
            pl.BlockSpec((D, 3 * U), lambda bi, ti: (0, 0)),
            pl.BlockSpec((U, 3 * U), lambda bi, ti: (0, 0)),
            pl.BlockSpec((2, 3 * U), lambda bi, ti: (0, 0)),
        ],
        out_specs=[
            pl.BlockSpec((TB, BB, U), lambda bi, ti: (ti, bi, 0)),
            pl.BlockSpec((BB, U), lambda bi, ti: (bi, 0)),
        ],
        out_shape=[
            jax.ShapeDtypeStruct((T, B, U), xt.dtype),
            jax.ShapeDtypeStruct((B, U), xt.dtype),
        ],
        scratch_shapes=[
            pltpu.VMEM((TB, BB, 3 * U), jnp.float32),
            pltpu.VMEM((BB, U), jnp.float32),
        ],
        compiler_params=pltpu.CompilerParams(
            dimension_semantics=("parallel", "arbitrary"),
        ),
        name="gru_layer",
        interpret=interpret,
    )(xt, w, rw, b)
    return yt, hT


def kernel(x, kernels, rec_kernels, biases, *, interpret=False):
    L = kernels.shape[0]
    seq = jnp.swapaxes(x, 0, 1)
    finals = []
    for l in range(L):
        seq, hT = _gru_layer(seq, kernels[l], rec_kernels[l], biases[l],
                             interpret=interpret)
        finals.append(hT)
    out = jnp.swapaxes(seq, 0, 1)
    return out, jnp.stack(finals)


# speedup vs baseline: 6.2789x; 1.5784x over previous
import functools

import jax
import jax.numpy as jnp
from jax.experimental import pallas as pl
from jax.experimental.pallas import tpu as pltpu


def _gru_layer_body(x_ref, w_ref, rw_ref, b_ref, y_ref, hT_ref, gx_ref, h_ref,
                    *, TB, BB, U):
    t = pl.program_id(1)

    @pl.when(t == 0)
    def _init():
        h_ref[...] = jnp.zeros_like(h_ref)

    D = x_ref.shape[-1]
    xb = x_ref[...].reshape(TB * BB, D)
    gx = jnp.dot(xb, w_ref[...], preferred_element_type=jnp.float32) + b_ref[0]
    gx_ref[...] = gx.reshape(TB, BB, 3 * U)
    rb = b_ref[1]

    def step(i, _):
        h = h_ref[...]
        gxt = gx_ref[i]
        gh = jnp.dot(h, rw_ref[...], preferred_element_type=jnp.float32) + rb
        z = jax.nn.sigmoid(gxt[:, :U] + gh[:, :U])
        r = jax.nn.sigmoid(gxt[:, U:2 * U] + gh[:, U:2 * U])
        hh = jnp.tanh(gxt[:, 2 * U:] + r * gh[:, 2 * U:])
        hn = z * h + (1.0 - z) * hh
        h_ref[...] = hn
        y_ref[i] = hn.astype(y_ref.dtype)
        return None

    jax.lax.fori_loop(0, TB, step, None)
    hT_ref[...] = h_ref[...].astype(hT_ref.dtype)


def _gru_layer(xt, w, rw, b, *, interpret=False):
    T, B, D = xt.shape
    U = rw.shape[0]
    NB = 1
    BB = B // NB
    TB = 16
    NT = T // TB

    body = functools.partial(_gru_layer_body, TB=TB, BB=BB, U=U)
    yt, hT = pl.pallas_call(
        body,
        grid=(NB, NT),
        in_specs=[
            pl.BlockSpec((TB, BB, D), lambda bi, ti: (ti, bi, 0)),
            pl.BlockSpec((D, 3 * U), lambda bi, ti: (0, 0)),
            pl.BlockSpec((U, 3 * U), lambda bi, ti: (0, 0)),
            pl.BlockSpec((2, 3 * U), lambda bi, ti: (0, 0)),
        ],
        out_specs=[
            pl.BlockSpec((TB, BB, U), lambda bi, ti: (ti, bi, 0)),
            pl.BlockSpec((BB, U), lambda bi, ti: (bi, 0)),
        ],
        out_shape=[
            jax.ShapeDtypeStruct((T, B, U), xt.dtype),
            jax.ShapeDtypeStruct((B, U), xt.dtype),
        ],
        scratch_shapes=[
            pltpu.VMEM((TB, BB, 3 * U), jnp.float32),
            pltpu.VMEM((BB, U), jnp.float32),
        ],
        compiler_params=pltpu.CompilerParams(
            dimension_semantics=("parallel", "arbitrary"),
        ),
        name="gru_layer",
        interpret=interpret,
    )(xt, w, rw, b)
    return yt, hT


def kernel(x, kernels, rec_kernels, biases, *, interpret=False):
    L = kernels.shape[0]
    seq = jnp.swapaxes(x, 0, 1)
    finals = []
    for l in range(L):
        seq, hT = _gru_layer(seq, kernels[l], rec_kernels[l], biases[l],
                             interpret=interpret)
        finals.append(hT)
    out = jnp.swapaxes(seq, 0, 1)
    return out, jnp.stack(finals)


# speedup vs baseline: 7.5503x; 1.2025x over previous
import functools

import jax
import jax.numpy as jnp
from jax.experimental import pallas as pl
from jax.experimental.pallas import tpu as pltpu

_TB = 8


def _wavefront_body(x_ref, w_ref, rw_ref, bgx_ref, brh_ref, y_ref, hT_ref,
                    gx_s, hb_s, h_s, *, L, TB, NT, B, U):
    s = pl.program_id(0)
    par = jax.lax.rem(s, 2)
    prev = 1 - par
    D = x_ref.shape[-1]

    for l in range(L):
        @pl.when(jnp.logical_and(s >= l, s <= NT - 1 + l))
        def _proj():
            if l == 0:
                src = x_ref[...].reshape(TB * B, D)
            else:
                src = hb_s[pl.ds((prev * (L - 1) + (l - 1)) * TB, TB)].reshape(
                    TB * B, U)
            g = jnp.dot(src, w_ref[l], preferred_element_type=jnp.float32)
            gx_s[l] = (g + bgx_ref[l]).reshape(TB, B, 3 * U)

        @pl.when(s == l)
        def _init():
            h_s[l] = jnp.zeros_like(h_s[l])

    def step(t, _):
        for l in range(L):
            h = h_s[l]
            gxt = gx_s.at[l][t]
            gh = jnp.dot(h.astype(jnp.bfloat16), rw_ref[l],
                         preferred_element_type=jnp.float32)
            z = jax.nn.sigmoid(gxt[:, :U] + gh[:, :U])
            r = jax.nn.sigmoid(gxt[:, U:2 * U] + gh[:, U:2 * U])
            rh = gh[:, 2 * U:] + brh_ref[l]
            hh = jnp.tanh(gxt[:, 2 * U:] + r * rh)
            hn = z * h + (1.0 - z) * hh
            h_s[l] = hn
            if l < L - 1:
                hb_s[(par * (L - 1) + l) * TB + t] = hn.astype(jnp.bfloat16)
            else:
                y_ref[t] = hn.astype(y_ref.dtype)
        return None

    jax.lax.fori_loop(0, TB, step, None)

    for l in range(L):
        @pl.when(s == NT - 1 + l)
        def _fin():
            hT_ref[l] = h_s[l].astype(hT_ref.dtype)


def kernel(x, kernels, rec_kernels, biases, *, interpret=False):
    B, T, D = x.shape
    L, _, threeU = kernels.shape
    U = threeU // 3
    TB = _TB
    NT = T // TB
    S = NT + L - 1

    xt = jnp.swapaxes(x, 0, 1).astype(jnp.bfloat16)
    w_bf = kernels.astype(jnp.bfloat16)
    rw_bf = rec_kernels.astype(jnp.bfloat16)
    b0 = biases[:, 0, :]
    b1 = biases[:, 1, :]
    b_gx = b0 + jnp.concatenate(
        [b1[:, :2 * U], jnp.zeros_like(b1[:, 2 * U:])], axis=-1)
    b_rh = b1[:, 2 * U:]

    body = functools.partial(_wavefront_body, L=L, TB=TB, NT=NT, B=B, U=U)
    yt, hT = pl.pallas_call(
        body,
        grid=(S,),
        in_specs=[
            pl.BlockSpec((TB, B, D), lambda s: (jnp.minimum(s, NT - 1), 0, 0)),
            pl.BlockSpec((L, D, 3 * U), lambda s: (0, 0, 0)),
            pl.BlockSpec((L, U, 3 * U), lambda s: (0, 0, 0)),
            pl.BlockSpec((L, 3 * U), lambda s: (0, 0)),
            pl.BlockSpec((L, U), lambda s: (0, 0)),
        ],
        out_specs=[
            pl.BlockSpec(
                (TB, B, U),
                lambda s: (jnp.clip(s - (L - 1), 0, NT - 1), 0, 0)),
            pl.BlockSpec((L, B, U), lambda s: (0, 0, 0)),
        ],
        out_shape=[
            jax.ShapeDtypeStruct((T, B, U), x.dtype),
            jax.ShapeDtypeStruct((L, B, U), x.dtype),
        ],
        scratch_shapes=[
            pltpu.VMEM((L, TB, B, 3 * U), jnp.float32),
            pltpu.VMEM((2 * (L - 1) * TB, B, U), jnp.bfloat16),
            pltpu.VMEM((L, B, U), jnp.float32),
        ],
        compiler_params=pltpu.CompilerParams(
            dimension_semantics=("arbitrary",),
        ),
        name="gru_wavefront",
        interpret=interpret,
    )(xt, w_bf, rw_bf, b_gx, b_rh)
    return jnp.swapaxes(yt, 0, 1), hT


# speedup vs baseline: 8.4933x; 1.1249x over previous
import functools

import jax
import jax.numpy as jnp
from jax.experimental import pallas as pl
from jax.experimental.pallas import tpu as pltpu

_TB = 8


def _wavefront_body(x_ref, w_ref, rw_ref, bgx_ref, brh_ref, y_ref, hT_ref,
                    gx_s, hb_s, h_s, *, L, TB, NT, B, U):
    s = pl.program_id(0)
    par = jax.lax.rem(s, 2)
    prev = 1 - par
    D = x_ref.shape[-1]

    for l in range(L):
        h_s[l] = jnp.where(s == l, jnp.zeros_like(h_s[l]), h_s[l])

    for l in range(L):
        if l == 0:
            src = x_ref[...].reshape(TB * B, D)
        else:
            src = hb_s[pl.ds((prev * (L - 1) + (l - 1)) * TB, TB)].reshape(
                TB * B, U)
        g = jnp.dot(src, w_ref[l], preferred_element_type=jnp.float32)
        gx_s[l] = (g + bgx_ref[l]).reshape(TB, B, 3 * U)

    for t in range(TB):
        for l in range(L):
            h = h_s[l]
            gxt = gx_s[l, t]
            gh = jnp.dot(h.astype(jnp.bfloat16), rw_ref[l],
                         preferred_element_type=jnp.float32)
            z = jax.nn.sigmoid(gxt[:, :U] + gh[:, :U])
            r = jax.nn.sigmoid(gxt[:, U:2 * U] + gh[:, U:2 * U])
            rh = gh[:, 2 * U:] + brh_ref[l]
            hh = jnp.tanh(gxt[:, 2 * U:] + r * rh)
            hn = z * h + (1.0 - z) * hh
            h_s[l] = hn
            if l < L - 1:
                hb_s[(par * (L - 1) + l) * TB + t] = hn.astype(jnp.bfloat16)
            else:
                y_ref[t] = hn.astype(y_ref.dtype)

    for l in range(L):
        fin = (s == NT - 1 + l)
        hT_ref[l] = jnp.where(fin, h_s[l], hT_ref[l]).astype(hT_ref.dtype)


def kernel(x, kernels, rec_kernels, biases, *, interpret=False):
    B, T, D = x.shape
    L, _, threeU = kernels.shape
    U = threeU // 3
    TB = _TB
    NT = T // TB
    S = NT + L - 1

    xt = jnp.swapaxes(x, 0, 1).astype(jnp.bfloat16)
    w_bf = kernels.astype(jnp.bfloat16)
    rw_bf = rec_kernels.astype(jnp.bfloat16)
    b0 = biases[:, 0, :]
    b1 = biases[:, 1, :]
    b_gx = b0 + jnp.concatenate(
        [b1[:, :2 * U], jnp.zeros_like(b1[:, 2 * U:])], axis=-1)
    b_rh = b1[:, 2 * U:]

    body = functools.partial(_wavefront_body, L=L, TB=TB, NT=NT, B=B, U=U)
    yt, hT = pl.pallas_call(
        body,
        grid=(S,),
        in_specs=[
            pl.BlockSpec((TB, B, D), lambda s: (jnp.minimum(s, NT - 1), 0, 0)),
            pl.BlockSpec((L, D, 3 * U), lambda s: (0, 0, 0)),
            pl.BlockSpec((L, U, 3 * U), lambda s: (0, 0, 0)),
            pl.BlockSpec((L, 3 * U), lambda s: (0, 0)),
            pl.BlockSpec((L, U), lambda s: (0, 0)),
        ],
        out_specs=[
            pl.BlockSpec(
                (TB, B, U),
                lambda s: (jnp.clip(s - (L - 1), 0, NT - 1), 0, 0)),
            pl.BlockSpec((L, B, U), lambda s: (0, 0, 0)),
        ],
        out_shape=[
            jax.ShapeDtypeStruct((T, B, U), x.dtype),
            jax.ShapeDtypeStruct((L, B, U), x.dtype),
        ],
        scratch_shapes=[
            pltpu.VMEM((L, TB, B, 3 * U), jnp.float32),
            pltpu.VMEM((2 * (L - 1) * TB, B, U), jnp.bfloat16),
            pltpu.VMEM((L, B, U), jnp.float32),
        ],
        compiler_params=pltpu.CompilerParams(
            dimension_semantics=("arbitrary",),
        ),
        name="gru_wavefront",
        interpret=interpret,
    )(xt, w_bf, rw_bf, b_gx, b_rh)
    return jnp.swapaxes(yt, 0, 1), hT
